# baseline (speedup 1.0000x reference)
import functools

import jax
import jax.numpy as jnp
from jax import lax
from jax.experimental import pallas as pl
from jax.experimental.pallas import tpu as pltpu
from jax.experimental.pallas import tpu_sc as plsc

_VOCAB = 100000
_EMBED = 64
_BATCH = 1024

_BV = 1024


def _make_sc_gather(B, D):
    info = plsc.get_sparse_core_info()
    nw = info.num_cores * info.num_subcores
    b_per_w = B // nw
    mesh = plsc.VectorSubcoreMesh(core_axis_name="c", subcore_axis_name="s")

    @functools.partial(
        pl.kernel,
        mesh=mesh,
        out_type=jax.ShapeDtypeStruct((B, D), jnp.float32),
        scratch_types=[
            pltpu.VMEM((b_per_w,), jnp.int32),
            pltpu.VMEM((b_per_w, D), jnp.float32),
            pltpu.SemaphoreType.DMA,
        ],
        compiler_params=pltpu.CompilerParams(use_tc_tiling_on_sc=False),
    )
    def gather_rows(idx_hbm, table_hbm, out_hbm, idx_v, rows_v, sem):
        wid = lax.axis_index("s") * info.num_cores + lax.axis_index("c")
        base = wid * b_per_w
        pltpu.sync_copy(idx_hbm.at[pl.ds(base, b_per_w)], idx_v)
        pltpu.async_copy(table_hbm.at[idx_v], rows_v, sem).wait()
        pltpu.sync_copy(rows_v, out_hbm.at[pl.ds(base, b_per_w)])

    return gather_rows


_sc_gather = _make_sc_gather(_BATCH, _EMBED)


_NB = 6
_STEPS = pl.cdiv(_VOCAB, _BV)
_TAIL = _VOCAB - (_STEPS - 1) * _BV
_TAIL_A = (_TAIL // 128) * 128
_TAIL_B = _TAIL - _TAIL_A


def _dot(a, b):
    return lax.dot_general(
        a, b,
        dimension_numbers=(((1,), (1,)), ((), ())),
        preferred_element_type=jnp.float32,
    )


def _mm_body(emb_ref, w_ref, out_hbm, scr, scrt, sems, semt):
    j = pl.program_id(0)
    jm = lax.rem(j, _NB)

    @pl.when(j >= _NB)
    def _wait_reuse():
        pltpu.make_async_copy(
            scr.at[jm], out_hbm.at[:, pl.ds(0, _BV)], sems.at[jm]
        ).wait()

    scr[jm] = _dot(emb_ref[...], w_ref[...])

    @pl.when(j < _STEPS - 1)
    def _start_full():
        pltpu.make_async_copy(
            scr.at[jm], out_hbm.at[:, pl.ds(j * _BV, _BV)], sems.at[jm]
        ).start()

    @pl.when(j == _STEPS - 1)
    def _tail_and_drain():
        tm = (_STEPS - 1) % _NB
        base = (_STEPS - 1) * _BV
        pltpu.make_async_copy(
            scr.at[tm, :, pl.ds(0, _TAIL_A)],
            out_hbm.at[:, pl.ds(base, _TAIL_A)],
            sems.at[tm],
        ).start()
        scrt[...] = _dot(emb_ref[...], w_ref[pl.ds(_TAIL_A, _TAIL_B), :])
        pltpu.make_async_copy(
            scrt, out_hbm.at[:, pl.ds(base + _TAIL_A, _TAIL_B)], semt
        ).start()
        pltpu.make_async_copy(
            scrt, out_hbm.at[:, pl.ds(base + _TAIL_A, _TAIL_B)], semt
        ).wait()
        for k in range(_NB):
            if k == tm:
                pltpu.make_async_copy(
                    scr.at[k, :, pl.ds(0, _TAIL_A)],
                    out_hbm.at[:, pl.ds(0, _TAIL_A)],
                    sems.at[k],
                ).wait()
            else:
                pltpu.make_async_copy(
                    scr.at[k], out_hbm.at[:, pl.ds(0, _BV)], sems.at[k]
                ).wait()


def _tc_project(emb, w):
    return pl.pallas_call(
        _mm_body,
        grid=(_STEPS,),
        in_specs=[
            pl.BlockSpec((_BATCH, _EMBED), lambda j: (0, 0)),
            pl.BlockSpec((_BV, _EMBED), lambda j: (j, 0)),
        ],
        out_specs=pl.BlockSpec(memory_space=pl.ANY),
        out_shape=jax.ShapeDtypeStruct((_BATCH, _VOCAB), jnp.float32),
        scratch_shapes=[
            pltpu.VMEM((_NB, _BATCH, _BV), jnp.float32),
            pltpu.VMEM((_BATCH, _TAIL_B), jnp.float32),
            pltpu.SemaphoreType.DMA((_NB,)),
            pltpu.SemaphoreType.DMA,
        ],
        compiler_params=pltpu.CompilerParams(
            dimension_semantics=("arbitrary",),
        ),
    )(emb, w)


def kernel(center, emb_table, w):
    emb = _sc_gather(center, emb_table)
    return _tc_project(emb, w)


# speedup vs baseline: 1.0161x; 1.0161x over previous
import functools

import jax
import jax.numpy as jnp
from jax import lax
from jax.experimental import pallas as pl
from jax.experimental.pallas import tpu as pltpu
from jax.experimental.pallas import tpu_sc as plsc

_VOCAB = 100000
_EMBED = 64
_BATCH = 1024

_BV = 1024
_NB = 6
_STEPS = pl.cdiv(_VOCAB, _BV)
_TAIL = _VOCAB - (_STEPS - 1) * _BV
_TAIL_A = (_TAIL // 128) * 128
_TAIL_B = _TAIL - _TAIL_A

_SLAB = 8


def _make_sc_gather(B, D):
    info = plsc.get_sparse_core_info()
    nw = info.num_cores * info.num_subcores
    b_per_w = B // nw
    mesh = plsc.VectorSubcoreMesh(core_axis_name="c", subcore_axis_name="s")

    @functools.partial(
        pl.kernel,
        mesh=mesh,
        out_type=jax.ShapeDtypeStruct((B, D), jnp.float32),
        scratch_types=[
            pltpu.VMEM((b_per_w,), jnp.int32),
            pltpu.VMEM((b_per_w, D), jnp.float32),
            pltpu.SemaphoreType.DMA,
        ],
    )
    def gather_rows(idx_hbm, table_hbm, out_hbm, idx_v, rows_v, sem):
        wid = lax.axis_index("s") * info.num_cores + lax.axis_index("c")
        base = wid * b_per_w
        pltpu.sync_copy(idx_hbm.at[pl.ds(base, b_per_w)], idx_v)
        pltpu.async_copy(table_hbm.at[idx_v], rows_v, sem).wait()
        pltpu.sync_copy(rows_v, out_hbm.at[pl.ds(base, b_per_w)])

    return gather_rows


_sc_gather = _make_sc_gather(_BATCH, 128)


def _dot(a, b):
    return lax.dot_general(
        a, b,
        dimension_numbers=(((1,), (1,)), ((), ())),
        preferred_element_type=jnp.float32,
    )


def _mm_body(emb_ref, w_ref, out_hbm, scr, scrt, sems, semt):
    j = pl.program_id(0)
    jm = lax.rem(j, _NB)

    @pl.when(j >= _NB)
    def _wait_reuse():
        pltpu.make_async_copy(
            scr.at[jm], out_hbm.at[:, pl.ds(0, _BV)], sems.at[jm]
        ).wait()

    scr[jm] = _dot(emb_ref[:, pl.ds(0, _EMBED)], w_ref[...])

    @pl.when(j < _STEPS - 1)
    def _start_full():
        pltpu.make_async_copy(
            scr.at[jm], out_hbm.at[:, pl.ds(j * _BV, _BV)], sems.at[jm]
        ).start()

    @pl.when(j == _STEPS - 1)
    def _tail_and_drain():
        tm = (_STEPS - 1) % _NB
        base = (_STEPS - 1) * _BV
        pltpu.make_async_copy(
            scr.at[tm, :, pl.ds(0, _TAIL_A)],
            out_hbm.at[:, pl.ds(base, _TAIL_A)],
            sems.at[tm],
        ).start()
        scrt[...] = _dot(emb_ref[:, pl.ds(0, _EMBED)], w_ref[pl.ds(_TAIL_A, _TAIL_B), :])
        pltpu.make_async_copy(
            scrt, out_hbm.at[:, pl.ds(base + _TAIL_A, _TAIL_B)], semt
        ).start()
        pltpu.make_async_copy(
            scrt, out_hbm.at[:, pl.ds(base + _TAIL_A, _TAIL_B)], semt
        ).wait()
        for k in range(_NB):
            if k == tm:
                pltpu.make_async_copy(
                    scr.at[k, :, pl.ds(0, _TAIL_A)],
                    out_hbm.at[:, pl.ds(0, _TAIL_A)],
                    sems.at[k],
                ).wait()
            else:
                pltpu.make_async_copy(
                    scr.at[k], out_hbm.at[:, pl.ds(0, _BV)], sems.at[k]
                ).wait()


def _tc_project(emb, w):
    return pl.pallas_call(
        _mm_body,
        grid=(_STEPS,),
        in_specs=[
            pl.BlockSpec((_BATCH, 128), lambda j: (0, 0)),
            pl.BlockSpec((_BV, _EMBED), lambda j: (j, 0)),
        ],
        out_specs=pl.BlockSpec(memory_space=pl.ANY),
        out_shape=jax.ShapeDtypeStruct((_BATCH, _VOCAB), jnp.float32),
        scratch_shapes=[
            pltpu.VMEM((_NB, _BATCH, _BV), jnp.float32),
            pltpu.VMEM((_BATCH, _TAIL_B), jnp.float32),
            pltpu.SemaphoreType.DMA((_NB,)),
            pltpu.SemaphoreType.DMA,
        ],
        compiler_params=pltpu.CompilerParams(
            dimension_semantics=("arbitrary",),
        ),
    )(emb, w)


def kernel(center, emb_table, w):
    table128 = jnp.pad(emb_table, ((0, 0), (0, 128 - _EMBED)))
    emb = _sc_gather(center, table128)
    return _tc_project(emb, w)


# speedup vs baseline: 1.0173x; 1.0011x over previous
import functools

import jax
import jax.numpy as jnp
from jax import lax
from jax.experimental import pallas as pl
from jax.experimental.pallas import tpu as pltpu
from jax.experimental.pallas import tpu_sc as plsc

_VOCAB = 100000
_EMBED = 64
_BATCH = 1024

_BV = 1024
_NB = 6
_STEPS = pl.cdiv(_VOCAB, _BV)
_TAIL = _VOCAB - (_STEPS - 1) * _BV
_TAIL_A = (_TAIL // 128) * 128
_TAIL_B = _TAIL - _TAIL_A

_SLAB = 8


def _make_sc_gather(B, D):
    info = plsc.get_sparse_core_info()
    nw = info.num_cores * info.num_subcores
    b_per_w = B // nw
    mesh = plsc.VectorSubcoreMesh(core_axis_name="c", subcore_axis_name="s")

    @functools.partial(
        pl.kernel,
        mesh=mesh,
        out_type=jax.ShapeDtypeStruct((B, D), jnp.float32),
        scratch_types=[
            pltpu.VMEM((b_per_w,), jnp.int32),
            pltpu.VMEM((b_per_w, D), jnp.float32),
            pltpu.SemaphoreType.DMA,
        ],
        compiler_params=pltpu.CompilerParams(use_tc_tiling_on_sc=True),
    )
    def gather_rows(idx_hbm, table_hbm, out_hbm, idx_v, rows_v, sem):
        wid = lax.axis_index("s") * info.num_cores + lax.axis_index("c")
        base = wid * b_per_w
        pltpu.sync_copy(idx_hbm.at[pl.ds(base, b_per_w)], idx_v)
        pltpu.async_copy(table_hbm.at[idx_v], rows_v, sem).wait()
        pltpu.sync_copy(rows_v, out_hbm.at[pl.ds(base, b_per_w)])

    return gather_rows


_sc_gather = _make_sc_gather(_BATCH, 128)


def _dot(a, b):
    return lax.dot_general(
        a, b,
        dimension_numbers=(((1,), (1,)), ((), ())),
        preferred_element_type=jnp.float32,
    )


def _mm_body(emb_ref, w_ref, out_hbm, scr, scrt, sems, semt):
    j = pl.program_id(0)
    jm = lax.rem(j, _NB)

    @pl.when(j >= _NB)
    def _wait_reuse():
        pltpu.make_async_copy(
            scr.at[jm], out_hbm.at[:, pl.ds(0, _BV)], sems.at[jm]
        ).wait()

    scr[jm] = _dot(emb_ref[:, pl.ds(0, _EMBED)], w_ref[...])

    @pl.when(j < _STEPS - 1)
    def _start_full():
        pltpu.make_async_copy(
            scr.at[jm], out_hbm.at[:, pl.ds(j * _BV, _BV)], sems.at[jm]
        ).start()

    @pl.when(j == _STEPS - 1)
    def _tail_and_drain():
        tm = (_STEPS - 1) % _NB
        base = (_STEPS - 1) * _BV
        pltpu.make_async_copy(
            scr.at[tm, :, pl.ds(0, _TAIL_A)],
            out_hbm.at[:, pl.ds(base, _TAIL_A)],
            sems.at[tm],
        ).start()
        scrt[...] = _dot(emb_ref[:, pl.ds(0, _EMBED)], w_ref[pl.ds(_TAIL_A, _TAIL_B), :])
        pltpu.make_async_copy(
            scrt, out_hbm.at[:, pl.ds(base + _TAIL_A, _TAIL_B)], semt
        ).start()
        pltpu.make_async_copy(
            scrt, out_hbm.at[:, pl.ds(base + _TAIL_A, _TAIL_B)], semt
        ).wait()
        for k in range(_NB):
            if k == tm:
                pltpu.make_async_copy(
                    scr.at[k, :, pl.ds(0, _TAIL_A)],
                    out_hbm.at[:, pl.ds(0, _TAIL_A)],
                    sems.at[k],
                ).wait()
            else:
                pltpu.make_async_copy(
                    scr.at[k], out_hbm.at[:, pl.ds(0, _BV)], sems.at[k]
                ).wait()


def _tc_project(emb, w):
    return pl.pallas_call(
        _mm_body,
        grid=(_STEPS,),
        in_specs=[
            pl.BlockSpec((_BATCH, 128), lambda j: (0, 0)),
            pl.BlockSpec((_BV, _EMBED), lambda j: (j, 0)),
        ],
        out_specs=pl.BlockSpec(memory_space=pl.ANY),
        out_shape=jax.ShapeDtypeStruct((_BATCH, _VOCAB), jnp.float32),
        scratch_shapes=[
            pltpu.VMEM((_NB, _BATCH, _BV), jnp.float32),
            pltpu.VMEM((_BATCH, _TAIL_B), jnp.float32),
            pltpu.SemaphoreType.DMA((_NB,)),
            pltpu.SemaphoreType.DMA,
        ],
        compiler_params=pltpu.CompilerParams(
            dimension_semantics=("arbitrary",),
        ),
    )(emb, w)


def kernel(center, emb_table, w):
    table128 = jnp.pad(emb_table, ((0, 0), (0, 128 - _EMBED)))
    emb = _sc_gather(center, table128)
    return _tc_project(emb, w)


# speedup vs baseline: 3.0210x; 2.9697x over previous
import functools

import jax
import jax.numpy as jnp
from jax import lax
from jax.experimental import pallas as pl
from jax.experimental.pallas import tpu as pltpu
from jax.experimental.pallas import tpu_sc as plsc

_VOCAB = 100000
_EMBED = 64
_BATCH = 1024

_BV = 1024
_NB = 6
_STEPS = pl.cdiv(_VOCAB, _BV)
_TAIL = _VOCAB - (_STEPS - 1) * _BV
_TAIL_A = (_TAIL // 128) * 128
_TAIL_B = _TAIL - _TAIL_A

_SLAB = 8


def _make_sc_gather(B, D):
    info = plsc.get_sparse_core_info()
    nw = info.num_cores * info.num_subcores
    b_per_w = B // nw
    mesh = plsc.VectorSubcoreMesh(core_axis_name="c", subcore_axis_name="s")

    @functools.partial(
        pl.kernel,
        mesh=mesh,
        out_type=jax.ShapeDtypeStruct((B, D), jnp.float32),
        scratch_types=[
            pltpu.VMEM((b_per_w,), jnp.int32),
            pltpu.VMEM((b_per_w, D), jnp.float32),
            pltpu.SemaphoreType.DMA,
        ],
        compiler_params=pltpu.CompilerParams(use_tc_tiling_on_sc=True),
    )
    def gather_rows(idx_hbm, table_hbm, out_hbm, idx_v, rows_v, sem):
        wid = lax.axis_index("s") * info.num_cores + lax.axis_index("c")
        base = wid * b_per_w
        pltpu.sync_copy(idx_hbm.at[pl.ds(base, b_per_w)], idx_v)
        pltpu.async_copy(table_hbm.at[idx_v], rows_v, sem).wait()
        pltpu.sync_copy(rows_v, out_hbm.at[pl.ds(base, b_per_w)])

    return gather_rows


_sc_gather = _make_sc_gather(_BATCH, 128)


def _dot(a, b):
    return lax.dot_general(
        a, b,
        dimension_numbers=(((1,), (1,)), ((), ())),
        preferred_element_type=jnp.float32,
    )


def _mm_body(emb_ref, wt_ref, out_hbm, embt_s, scr, sems):
    j = pl.program_id(0)
    jm = lax.rem(j, _NB)

    @pl.when(j == 0)
    def _tr():
        embt_s[...] = emb_ref[:, pl.ds(0, _EMBED)].T

    @pl.when(j >= _NB)
    def _wait_reuse():
        pltpu.make_async_copy(
            scr.at[jm], out_hbm.at[pl.ds(0, _BV), :], sems.at[jm]
        ).wait()

    scr[jm] = lax.dot_general(
        wt_ref[...], embt_s[...],
        dimension_numbers=(((0,), (0,)), ((), ())),
        preferred_element_type=jnp.float32,
    )

    @pl.when(j < _STEPS - 1)
    def _start_full():
        pltpu.make_async_copy(
            scr.at[jm], out_hbm.at[pl.ds(j * _BV, _BV), :], sems.at[jm]
        ).start()

    @pl.when(j == _STEPS - 1)
    def _tail_and_drain():
        tm = (_STEPS - 1) % _NB
        base = (_STEPS - 1) * _BV
        pltpu.make_async_copy(
            scr.at[tm, pl.ds(0, _TAIL), :],
            out_hbm.at[pl.ds(base, _TAIL), :],
            sems.at[tm],
        ).start()
        for k in range(_NB):
            if k == tm:
                pltpu.make_async_copy(
                    scr.at[k, pl.ds(0, _TAIL), :],
                    out_hbm.at[pl.ds(0, _TAIL), :],
                    sems.at[k],
                ).wait()
            else:
                pltpu.make_async_copy(
                    scr.at[k], out_hbm.at[pl.ds(0, _BV), :], sems.at[k]
                ).wait()


def _tc_project(emb, wt):
    return pl.pallas_call(
        _mm_body,
        grid=(_STEPS,),
        in_specs=[
            pl.BlockSpec((_BATCH, 128), lambda j: (0, 0)),
            pl.BlockSpec((_EMBED, _BV), lambda j: (0, j)),
        ],
        out_specs=pl.BlockSpec(memory_space=pl.ANY),
        out_shape=jax.ShapeDtypeStruct((_VOCAB, _BATCH), jnp.float32),
        scratch_shapes=[
            pltpu.VMEM((_EMBED, _BATCH), jnp.float32),
            pltpu.VMEM((_NB, _BV, _BATCH), jnp.float32),
            pltpu.SemaphoreType.DMA((_NB,)),
        ],
        compiler_params=pltpu.CompilerParams(
            dimension_semantics=("arbitrary",),
            fuse_transposed_lhs_in_matmul=True,
        ),
    )(emb, wt)


_BT = 2048


def _tp_body(tt_ref, out_ref):
    out_ref[:, pl.ds(0, _EMBED)] = tt_ref[...].T
    out_ref[:, pl.ds(_EMBED, 128 - _EMBED)] = jnp.zeros(
        (_BT, 128 - _EMBED), jnp.float32
    )


def _tc_transpose_pad(tt):
    return pl.pallas_call(
        _tp_body,
        grid=(pl.cdiv(_VOCAB, _BT),),
        in_specs=[pl.BlockSpec((_EMBED, _BT), lambda j: (0, j))],
        out_specs=pl.BlockSpec((_BT, 128), lambda j: (j, 0)),
        out_shape=jax.ShapeDtypeStruct((_VOCAB, 128), jnp.float32),
    )(tt)


def kernel(center, emb_table, w):
    table128 = _tc_transpose_pad(emb_table.T)
    emb = _sc_gather(center, table128)
    outT = _tc_project(emb, w.T)
    return outT.T


# speedup vs baseline: 3.0320x; 1.0037x over previous
import functools

import jax
import jax.numpy as jnp
from jax import lax
from jax.experimental import pallas as pl
from jax.experimental.pallas import tpu as pltpu
from jax.experimental.pallas import tpu_sc as plsc

_VOCAB = 100000
_EMBED = 64
_BATCH = 1024

_BV = 1024
_NB = 6
_STEPS = pl.cdiv(_VOCAB, _BV)
_TAIL = _VOCAB - (_STEPS - 1) * _BV

def _make_sc_gather(B, D):
    info = plsc.get_sparse_core_info()
    nw = info.num_cores * info.num_subcores
    b_per_w = B // nw
    mesh = plsc.VectorSubcoreMesh(core_axis_name="c", subcore_axis_name="s")

    @functools.partial(
        pl.kernel,
        mesh=mesh,
        out_type=jax.ShapeDtypeStruct((B, D), jnp.float32),
        scratch_types=[
            pltpu.VMEM((b_per_w,), jnp.int32),
            pltpu.VMEM((b_per_w,), jnp.int32),
            pltpu.VMEM((b_per_w, D), jnp.float32),
            pltpu.SemaphoreType.DMA,
        ],
        compiler_params=pltpu.CompilerParams(use_tc_tiling_on_sc=True),
    )
    def gather_rows(idx_hbm, table_hbm, out_hbm, idx_v, pair_v, rows_v, sem):
        wid = lax.axis_index("s") * info.num_cores + lax.axis_index("c")
        base = wid * b_per_w
        pltpu.sync_copy(idx_hbm.at[pl.ds(base, b_per_w)], idx_v)
        for i in range(b_per_w // 16):
            sl = pl.ds(i * 16, 16)
            v = idx_v[sl]
            pair_v[sl] = (
                lax.shift_left(lax.shift_right_logical(v, 11), 10)
                | (v & 1023)
            )
        pltpu.async_copy(table_hbm.at[pair_v], rows_v, sem).wait()
        pltpu.sync_copy(rows_v, out_hbm.at[pl.ds(base, b_per_w)])

    return gather_rows


_sc_gather = _make_sc_gather(_BATCH, 128)

_PAIRS = 50176


def _mm_body(ctr_ref, emb_ref, wt_ref, out_hbm, embt_s, scr, sems):
    j = pl.program_id(0)
    jm = lax.rem(j, _NB)

    @pl.when(j == 0)
    def _tr():
        par = (lax.shift_right_logical(ctr_ref[...], 10) & 1).astype(
            jnp.float32
        )
        emb64 = (emb_ref[:, pl.ds(0, _EMBED)] * (1.0 - par)
                 + emb_ref[:, pl.ds(_EMBED, _EMBED)] * par)
        embt_s[...] = emb64.T

    @pl.when(j >= _NB)
    def _wait_reuse():
        pltpu.make_async_copy(
            scr.at[jm], out_hbm.at[pl.ds(0, _BV), :], sems.at[jm]
        ).wait()

    scr[jm] = lax.dot_general(
        wt_ref[...], embt_s[...],
        dimension_numbers=(((0,), (0,)), ((), ())),
        preferred_element_type=jnp.float32,
    )

    @pl.when(j < _STEPS - 1)
    def _start_full():
        pltpu.make_async_copy(
            scr.at[jm], out_hbm.at[pl.ds(j * _BV, _BV), :], sems.at[jm]
        ).start()

    @pl.when(j == _STEPS - 1)
    def _tail_and_drain():
        tm = (_STEPS - 1) % _NB
        base = (_STEPS - 1) * _BV
        pltpu.make_async_copy(
            scr.at[tm, pl.ds(0, _TAIL), :],
            out_hbm.at[pl.ds(base, _TAIL), :],
            sems.at[tm],
        ).start()
        for k in range(_NB):
            if k == tm:
                pltpu.make_async_copy(
                    scr.at[k, pl.ds(0, _TAIL), :],
                    out_hbm.at[pl.ds(0, _TAIL), :],
                    sems.at[k],
                ).wait()
            else:
                pltpu.make_async_copy(
                    scr.at[k], out_hbm.at[pl.ds(0, _BV), :], sems.at[k]
                ).wait()


def _tc_project(ctr, emb, wt):
    return pl.pallas_call(
        _mm_body,
        grid=(_STEPS,),
        in_specs=[
            pl.BlockSpec((_BATCH, 1), lambda j: (0, 0)),
            pl.BlockSpec((_BATCH, 128), lambda j: (0, 0)),
            pl.BlockSpec((_EMBED, _BV), lambda j: (0, j)),
        ],
        out_specs=pl.BlockSpec(memory_space=pl.ANY),
        out_shape=jax.ShapeDtypeStruct((_VOCAB, _BATCH), jnp.float32),
        scratch_shapes=[
            pltpu.VMEM((_EMBED, _BATCH), jnp.float32),
            pltpu.VMEM((_NB, _BV, _BATCH), jnp.float32),
            pltpu.SemaphoreType.DMA((_NB,)),
        ],
        compiler_params=pltpu.CompilerParams(
            dimension_semantics=("arbitrary",),
            fuse_transposed_lhs_in_matmul=True,
        ),
    )(ctr, emb, wt)


_BT = 2048


def _tp_body(ta_ref, tb_ref, out_ref):
    out_ref[:, pl.ds(0, _EMBED)] = ta_ref[...].T
    out_ref[:, pl.ds(_EMBED, _EMBED)] = tb_ref[...].T


def _tc_transpose_pack(tt):
    return pl.pallas_call(
        _tp_body,
        grid=(_PAIRS // 1024,),
        in_specs=[
            pl.BlockSpec((_EMBED, 1024), lambda j: (0, 2 * j)),
            pl.BlockSpec((_EMBED, 1024), lambda j: (0, 2 * j + 1)),
        ],
        out_specs=pl.BlockSpec((1024, 128), lambda j: (j, 0)),
        out_shape=jax.ShapeDtypeStruct((_PAIRS, 128), jnp.float32),
    )(tt, tt)


def kernel(center, emb_table, w):
    pairs = _tc_transpose_pack(emb_table.T)
    emb = _sc_gather(center, pairs)
    outT = _tc_project(center.reshape(_BATCH, 1), emb, w.T)
    return outT.T


# speedup vs baseline: 3.0328x; 1.0003x over previous
import functools

import jax
import jax.numpy as jnp
from jax import lax
from jax.experimental import pallas as pl
from jax.experimental.pallas import tpu as pltpu
from jax.experimental.pallas import tpu_sc as plsc

_VOCAB = 100000
_EMBED = 64
_BATCH = 1024

_BV = 1024
_NB = 6
_STEPS = pl.cdiv(_VOCAB, _BV)
_TAIL = _VOCAB - (_STEPS - 1) * _BV

def _make_sc_gather(B, D):
    info = plsc.get_sparse_core_info()
    nw = info.num_cores * info.num_subcores
    b_per_w = B // nw
    mesh = plsc.VectorSubcoreMesh(core_axis_name="c", subcore_axis_name="s")

    @functools.partial(
        pl.kernel,
        mesh=mesh,
        out_type=jax.ShapeDtypeStruct((B, D), jnp.float32),
        scratch_types=[
            pltpu.VMEM((b_per_w,), jnp.int32),
            pltpu.VMEM((b_per_w,), jnp.int32),
            pltpu.VMEM((b_per_w, D), jnp.float32),
            pltpu.SemaphoreType.DMA,
        ],
        compiler_params=pltpu.CompilerParams(use_tc_tiling_on_sc=True),
    )
    def gather_rows(idx_hbm, table_hbm, out_hbm, idx_v, pair_v, rows_v, sem):
        wid = lax.axis_index("s") * info.num_cores + lax.axis_index("c")
        base = wid * b_per_w
        pltpu.sync_copy(idx_hbm.at[pl.ds(base, b_per_w)], idx_v)
        for i in range(b_per_w // 16):
            sl = pl.ds(i * 16, 16)
            v = idx_v[sl]
            pair_v[sl] = (
                lax.shift_left(lax.shift_right_logical(v, 11), 10)
                | (v & 1023)
            )
        pltpu.async_copy(table_hbm.at[pair_v], rows_v, sem).wait()
        pltpu.sync_copy(rows_v, out_hbm.at[pl.ds(base, b_per_w)])

    return gather_rows


_sc_gather = _make_sc_gather(_BATCH, 128)

_PAIRS = 50176


def _mm_body(ctr_ref, emb_ref, wt_ref, out_hbm, embt_s, scr, sems):
    j = pl.program_id(0)
    jm = lax.rem(j, _NB)

    @pl.when(j == 0)
    def _tr():
        par = (lax.shift_right_logical(ctr_ref[...], 10) & 1).astype(
            jnp.float32
        )
        emb64 = (emb_ref[:, pl.ds(0, _EMBED)] * (1.0 - par)
                 + emb_ref[:, pl.ds(_EMBED, _EMBED)] * par)
        embt_s[...] = emb64.T

    @pl.when(j >= _NB)
    def _wait_reuse():
        pltpu.make_async_copy(
            scr.at[jm], out_hbm.at[pl.ds(0, _BV), :], sems.at[jm]
        ).wait()

    scr[jm] = lax.dot_general(
        wt_ref[...], embt_s[...],
        dimension_numbers=(((0,), (0,)), ((), ())),
        preferred_element_type=jnp.float32,
    )

    @pl.when(j < _STEPS - 1)
    def _start_full():
        pltpu.make_async_copy(
            scr.at[jm], out_hbm.at[pl.ds(j * _BV, _BV), :], sems.at[jm]
        ).start()

    @pl.when(j == _STEPS - 1)
    def _tail_and_drain():
        tm = (_STEPS - 1) % _NB
        base = (_STEPS - 1) * _BV
        pltpu.make_async_copy(
            scr.at[tm, pl.ds(0, _TAIL), :],
            out_hbm.at[pl.ds(base, _TAIL), :],
            sems.at[tm],
        ).start()
        for k in range(_NB):
            if k == tm:
                pltpu.make_async_copy(
                    scr.at[k, pl.ds(0, _TAIL), :],
                    out_hbm.at[pl.ds(0, _TAIL), :],
                    sems.at[k],
                ).wait()
            else:
                pltpu.make_async_copy(
                    scr.at[k], out_hbm.at[pl.ds(0, _BV), :], sems.at[k]
                ).wait()


def _tc_project(ctr, emb, wt):
    return pl.pallas_call(
        _mm_body,
        grid=(_STEPS,),
        in_specs=[
            pl.BlockSpec((_BATCH, 1), lambda j: (0, 0)),
            pl.BlockSpec((_BATCH, 128), lambda j: (0, 0)),
            pl.BlockSpec((_EMBED, _BV), lambda j: (0, j)),
        ],
        out_specs=pl.BlockSpec(memory_space=pl.ANY),
        out_shape=jax.ShapeDtypeStruct((_VOCAB, _BATCH), jnp.float32),
        scratch_shapes=[
            pltpu.VMEM((_EMBED, _BATCH), jnp.float32),
            pltpu.VMEM((_NB, _BV, _BATCH), jnp.float32),
            pltpu.SemaphoreType.DMA((_NB,)),
        ],
        compiler_params=pltpu.CompilerParams(
            dimension_semantics=("arbitrary",),
            fuse_transposed_lhs_in_matmul=True,
        ),
    )(ctr, emb, wt)


_BT = 2048


_NPB = 4
_PSTEPS = _PAIRS // 1024


def _tp_body(ta_ref, tb_ref, out_hbm, scr, sems):
    j = pl.program_id(0)
    jm = lax.rem(j, _NPB)

    @pl.when(j >= _NPB)
    def _wait_reuse():
        pltpu.make_async_copy(
            scr.at[jm], out_hbm.at[pl.ds(0, 1024), :], sems.at[jm]
        ).wait()

    scr[jm, :, pl.ds(0, _EMBED)] = ta_ref[...].T
    scr[jm, :, pl.ds(_EMBED, _EMBED)] = tb_ref[...].T
    pltpu.make_async_copy(
        scr.at[jm], out_hbm.at[pl.ds(j * 1024, 1024), :], sems.at[jm]
    ).start()

    @pl.when(j == _PSTEPS - 1)
    def _drain():
        for k in range(_NPB):
            pltpu.make_async_copy(
                scr.at[k], out_hbm.at[pl.ds(0, 1024), :], sems.at[k]
            ).wait()


def _tc_transpose_pack(tt):
    return pl.pallas_call(
        _tp_body,
        grid=(_PSTEPS,),
        in_specs=[
            pl.BlockSpec((_EMBED, 1024), lambda j: (0, 2 * j)),
            pl.BlockSpec((_EMBED, 1024), lambda j: (0, 2 * j + 1)),
        ],
        out_specs=pl.BlockSpec(memory_space=pl.ANY),
        out_shape=jax.ShapeDtypeStruct((_PAIRS, 128), jnp.float32),
        scratch_shapes=[
            pltpu.VMEM((_NPB, 1024, 128), jnp.float32),
            pltpu.SemaphoreType.DMA((_NPB,)),
        ],
        compiler_params=pltpu.CompilerParams(
            dimension_semantics=("arbitrary",),
        ),
    )(tt, tt)


def kernel(center, emb_table, w):
    pairs = _tc_transpose_pack(emb_table.T)
    emb = _sc_gather(center, pairs)
    outT = _tc_project(center.reshape(_BATCH, 1), emb, w.T)
    return outT.T
